# baseline (speedup 1.0000x reference)
import functools

import jax
import jax.numpy as jnp
from jax import lax
from jax.experimental import pallas as pl
from jax.experimental.pallas import tpu as pltpu
from jax.experimental.pallas import tpu_sc as plsc

B, N, EPN, D = 8, 512, 32, 256
STEPS = 5

_NC, _NS = 2, 16
_NW = _NC * _NS
_RC = (B * N) // _NW
_L = 16


_CH = 32
_WPG = N // _RC


def _sc_build_body(an_hbm, ae_hbm, mcat_hbm, idxn_v, idxe_v,
                   buf0_v, buf1_v, sem0, sem1):
    wid = lax.axis_index("s") * _NC + lax.axis_index("c")
    base = wid * _RC
    b = wid // _WPG
    r0 = (wid % _WPG) * _RC
    pltpu.sync_copy(an_hbm.at[b, pl.ds(r0, _RC)], idxn_v)
    pltpu.sync_copy(ae_hbm.at[b, pl.ds(r0, _RC)], idxe_v)

    zero16 = jnp.zeros((_L,), jnp.float32)

    def zrow(t, carry):
        i = t // (3 * N // _L)
        j = t % (3 * N // _L)
        buf0_v[i, pl.ds(j * _L, _L)] = zero16
        buf1_v[i, pl.ds(j * _L, _L)] = zero16
        return carry

    lax.fori_loop(0, _CH * (3 * N // _L), zrow, 0)

    lane = lax.iota(jnp.int32, _L)
    ones = jnp.full((_L,), 1.0, jnp.float32)
    negs = jnp.full((_L,), -1.0, jnp.float32)
    bufs = (buf0_v, buf1_v)
    sems = (sem0, sem1)

    def scatter(buf, row_off, val):
        def body(t, carry):
            g = t // EPN
            k = t % EPN
            lrow = lane + g * _L
            srow = lrow + row_off
            kk = jnp.full((_L,), 1, jnp.int32) * k
            vn = plsc.load_gather(idxn_v, [srow, kk])
            plsc.addupdate_scatter(buf, [lrow, vn], val, mask=vn != 0)
            ve = plsc.load_gather(idxe_v, [srow, kk])
            plsc.addupdate_scatter(buf, [lrow, ve + N], val,
                                   mask=ve != 0)
            return carry
        lax.fori_loop(0, (_CH // _L) * EPN, body, 0)

    nchunks = _RC // _CH
    copies = [None] * nchunks
    for c in range(nchunks):
        buf = bufs[c % 2]
        if c >= 2:
            copies[c - 2].wait()
            scatter(buf, (c - 2) * _CH, negs)
        scatter(buf, c * _CH, ones)
        copies[c] = pltpu.async_copy(
            buf, mcat_hbm.at[pl.ds(base + c * _CH, _CH)], sems[c % 2])
    copies[nchunks - 2].wait()
    copies[nchunks - 1].wait()


_sc_build = functools.partial(
    pl.kernel,
    out_type=jax.ShapeDtypeStruct((B * N, 3 * N), jnp.float32),
    mesh=plsc.VectorSubcoreMesh(core_axis_name="c", subcore_axis_name="s"),
    compiler_params=pltpu.CompilerParams(needs_layout_passes=False),
    scratch_types=[
        pltpu.VMEM((_RC, EPN), jnp.int32),
        pltpu.VMEM((_RC, EPN), jnp.int32),
        pltpu.VMEM((_CH, 3 * N), jnp.float32),
        pltpu.VMEM((_CH, 3 * N), jnp.float32),
        pltpu.SemaphoreType.DMA,
        pltpu.SemaphoreType.DMA,
    ],
)(_sc_build_body)


def _gnn_body(nm_ref, em_ref, an_ref, ae_ref, mcat_ref,
              Wn_ref, bn_ref, We_ref, be_ref, Wout_ref, Win_ref, Wu_ref,
              bu_ref, Wr_ref, br_ref, Wh_ref, bh_ref, out_ref):
    f32 = jnp.float32
    bf16 = jnp.bfloat16
    An = an_ref[0]
    Ae = ae_ref[0]
    Mcat = mcat_ref[...].astype(bf16)
    Mn = Mcat[:, :N]
    Me = Mcat[:, N:]

    nodes_mask = (jnp.sum(An, axis=1) != 0).astype(f32)[:, None]
    edges_mask = (jnp.sum(Ae, axis=1) != 0).astype(f32)[:, None]

    def mm(a, b):
        return jnp.dot(a.astype(bf16), b.astype(bf16),
                       preferred_element_type=f32)

    nm = nm_ref[0]
    em = em_ref[0]
    S = jnp.tanh(mm(nm, Wn_ref[...]) + bn_ref[...]) * nodes_mask
    row_iota = jax.lax.broadcasted_iota(jnp.int32, (N, 1), 0)
    S = jnp.where(row_iota == 1, 1.0, S)

    e = jnp.tanh(mm(em, We_ref[...]) + be_ref[...]) * edges_mask
    ecat = jnp.concatenate([mm(e, Wout_ref[...]), mm(e, Win_ref[...])],
                           axis=0)
    act_e = mm(Me, ecat)

    Wu_a, Wu_s = Wu_ref[:D, :], Wu_ref[D:, :]
    Wr_a, Wr_s = Wr_ref[:D, :], Wr_ref[D:, :]
    Wh_a, Wh_s = Wh_ref[:D, :], Wh_ref[D:, :]
    bu = bu_ref[...]
    br = br_ref[...]
    bh = bh_ref[...]

    def sigmoid(x):
        return 0.5 * jnp.tanh(0.5 * x) + 0.5

    for _ in range(STEPS):
        act = mm(Mn, S) + act_e
        u = sigmoid(mm(act, Wu_a) + mm(S, Wu_s) + bu)
        r = sigmoid(mm(act, Wr_a) + mm(S, Wr_s) + br)
        h = jnp.tanh(mm(act, Wh_a) + mm(r * S, Wh_s) + bh)
        S = S + u * (h - S)

    out_ref[...] = S[1, :][None, None, :]


def kernel(nodes_m, edges_m, A_nodes, A_edges, Wn, bn, We, be, Wout, Win,
           Wu, bu, Wr, br, Wh, bh):
    Mcat = _sc_build(A_nodes, A_edges)

    bn2, be2, bu2, br2, bh2 = (x.reshape(1, D) for x in (bn, be, bu, br, bh))
    full2 = lambda shape: pl.BlockSpec(shape, lambda b: (0,) * len(shape))
    per_b3 = lambda d1, d2: pl.BlockSpec((1, d1, d2), lambda b: (b, 0, 0))
    return pl.pallas_call(
        _gnn_body,
        grid=(B,),
        in_specs=[
            per_b3(N, D),
            per_b3(N, D),
            per_b3(N, EPN),
            per_b3(N, EPN),
            pl.BlockSpec((N, 3 * N), lambda b: (b, 0)),
            full2((D, D)),
            full2((1, D)),
            full2((D, D)),
            full2((1, D)),
            full2((D, D)),
            full2((D, D)),
            full2((2 * D, D)),
            full2((1, D)),
            full2((2 * D, D)),
            full2((1, D)),
            full2((2 * D, D)),
            full2((1, D)),
        ],
        out_specs=pl.BlockSpec((1, 1, D), lambda b: (b, 0, 0)),
        out_shape=jax.ShapeDtypeStruct((B, 1, D), jnp.float32),
        compiler_params=pltpu.CompilerParams(
            dimension_semantics=("arbitrary",)),
    )(nodes_m, edges_m, A_nodes, A_edges, Mcat, Wn, bn2, We, be2,
      Wout, Win, Wu, bu2, Wr, br2, Wh, bh2).reshape(B, D)


# speedup vs baseline: 1.1262x; 1.1262x over previous
import functools

import jax
import jax.numpy as jnp
from jax import lax
from jax.experimental import pallas as pl
from jax.experimental.pallas import tpu as pltpu
from jax.experimental.pallas import tpu_sc as plsc

B, N, EPN, D = 8, 512, 32, 256
STEPS = 5

_NC, _NS = 2, 16
_NW = _NC * _NS
_RC = (B * N) // _NW
_L = 16


def _sc_build_body(anT_hbm, aeT_hbm, mn_hbm, meo_hbm, mei_hbm,
                   idxn_v, idxe_v, buf_v):
    wid = lax.axis_index("s") * _NC + lax.axis_index("c")
    base = wid * _RC
    pltpu.sync_copy(anT_hbm.at[:, pl.ds(base, _RC)], idxn_v)
    pltpu.sync_copy(aeT_hbm.at[:, pl.ds(base, _RC)], idxe_v)

    zero16 = jnp.zeros((_L,), jnp.float32)

    def zrow(i, carry):
        for j in range(N // _L):
            buf_v[i, pl.ds(j * _L, _L)] = zero16
        return carry

    lax.fori_loop(0, _RC, zrow, 0)

    lane = lax.iota(jnp.int32, _L)

    def scatter(idx_v, val, kind):
        def body(g, carry):
            row = lane + g * _L
            for k in range(EPN):
                idx = idx_v[k, pl.ds(g * _L, _L)]
                if kind == 0:
                    mask = (idx != 0) & (idx < N)
                    col = idx
                else:
                    mask = idx >= N
                    col = idx - N
                plsc.addupdate_scatter(buf_v, [row, col], val, mask=mask)
            return carry
        lax.fori_loop(0, _RC // _L, body, 0)

    ones = jnp.full((_L,), 1.0, jnp.float32)
    negs = jnp.full((_L,), -1.0, jnp.float32)

    scatter(idxn_v, ones, 0)
    pltpu.sync_copy(buf_v, mn_hbm.at[pl.ds(base, _RC)])
    scatter(idxn_v, negs, 0)

    scatter(idxe_v, ones, 0)
    pltpu.sync_copy(buf_v, meo_hbm.at[pl.ds(base, _RC)])
    scatter(idxe_v, negs, 0)

    scatter(idxe_v, ones, 1)
    pltpu.sync_copy(buf_v, mei_hbm.at[pl.ds(base, _RC)])


_sc_build = functools.partial(
    pl.kernel,
    out_type=(jax.ShapeDtypeStruct((B * N, N), jnp.float32),) * 3,
    mesh=plsc.VectorSubcoreMesh(core_axis_name="c", subcore_axis_name="s"),
    compiler_params=pltpu.CompilerParams(needs_layout_passes=False),
    scratch_types=[
        pltpu.VMEM((EPN, _RC), jnp.int32),
        pltpu.VMEM((EPN, _RC), jnp.int32),
        pltpu.VMEM((_RC, N), jnp.float32),
    ],
)(_sc_build_body)


def _proj_body(nm_ref, em_ref, an_ref, ae_ref, Wn_ref, bn_ref, We_ref,
               be_ref, Wout_ref, Win_ref, s0_ref, oe_ref, ie_ref):
    f32 = jnp.float32
    bf16 = jnp.bfloat16
    An = an_ref[0]
    Ae = ae_ref[0]
    nodes_mask = (jnp.sum(An, axis=1) != 0).astype(f32)[:, None]
    edges_mask = (jnp.sum(Ae, axis=1) != 0).astype(f32)[:, None]

    def mm(a, b):
        return jnp.dot(a.astype(bf16), b.astype(bf16),
                       preferred_element_type=f32)

    S = jnp.tanh(mm(nm_ref[0], Wn_ref[...]) + bn_ref[...]) * nodes_mask
    row_iota = jax.lax.broadcasted_iota(jnp.int32, (N, 1), 0)
    S = jnp.where(row_iota == 1, 1.0, S)
    e = jnp.tanh(mm(em_ref[0], We_ref[...]) + be_ref[...]) * edges_mask
    s0_ref[...] = S[None]
    oe_ref[...] = mm(e, Wout_ref[...])[None]
    ie_ref[...] = mm(e, Win_ref[...])[None]


def _gru_body(s0_ref, oe_ref, ie_ref, mn_ref, meo_ref, mei_ref,
              Wu_ref, bu_ref, Wr_ref, br_ref, Wh_ref, bh_ref, out_ref):
    f32 = jnp.float32
    bf16 = jnp.bfloat16
    Mn = mn_ref[...].astype(bf16)
    Meo = meo_ref[...].astype(bf16)
    Mei = mei_ref[...].astype(bf16)

    def mm(a, b):
        return jnp.dot(a.astype(bf16), b.astype(bf16),
                       preferred_element_type=f32)

    S = s0_ref[0]
    act_e = mm(Meo, oe_ref[0]) + mm(Mei, ie_ref[0])

    Wu_a, Wu_s = Wu_ref[:D, :], Wu_ref[D:, :]
    Wr_a, Wr_s = Wr_ref[:D, :], Wr_ref[D:, :]
    Wh_a, Wh_s = Wh_ref[:D, :], Wh_ref[D:, :]
    bu = bu_ref[...]
    br = br_ref[...]
    bh = bh_ref[...]

    def sigmoid(x):
        return 0.5 * jnp.tanh(0.5 * x) + 0.5

    for _ in range(STEPS):
        act = mm(Mn, S) + act_e
        u = sigmoid(mm(act, Wu_a) + mm(S, Wu_s) + bu)
        r = sigmoid(mm(act, Wr_a) + mm(S, Wr_s) + br)
        h = jnp.tanh(mm(act, Wh_a) + mm(r * S, Wh_s) + bh)
        S = S + u * (h - S)

    out_ref[...] = S[1, :][None, None, :]


def kernel(nodes_m, edges_m, A_nodes, A_edges, Wn, bn, We, be, Wout, Win,
           Wu, bu, Wr, br, Wh, bh):
    anT = A_nodes.reshape(B * N, EPN).T
    aeT = A_edges.reshape(B * N, EPN).T
    Mn, Meo, Mei = _sc_build(anT, aeT)

    bn2, be2, bu2, br2, bh2 = (x.reshape(1, D) for x in (bn, be, bu, br, bh))
    full2 = lambda shape: pl.BlockSpec(shape, lambda b: (0,) * len(shape))
    per_b3 = lambda d1, d2: pl.BlockSpec((1, d1, d2), lambda b: (b, 0, 0))
    m_spec = pl.BlockSpec((N, N), lambda b: (b, 0))

    S0, oe, ie = pl.pallas_call(
        _proj_body,
        grid=(B,),
        in_specs=[
            per_b3(N, D),
            per_b3(N, D),
            per_b3(N, EPN),
            per_b3(N, EPN),
            full2((D, D)),
            full2((1, D)),
            full2((D, D)),
            full2((1, D)),
            full2((D, D)),
            full2((D, D)),
        ],
        out_specs=[per_b3(N, D)] * 3,
        out_shape=[jax.ShapeDtypeStruct((B, N, D), jnp.float32)] * 3,
        compiler_params=pltpu.CompilerParams(
            dimension_semantics=("arbitrary",)),
    )(nodes_m, edges_m, A_nodes, A_edges, Wn, bn2, We, be2, Wout, Win)

    return pl.pallas_call(
        _gru_body,
        grid=(B,),
        in_specs=[
            per_b3(N, D),
            per_b3(N, D),
            per_b3(N, D),
            m_spec,
            m_spec,
            m_spec,
            full2((2 * D, D)),
            full2((1, D)),
            full2((2 * D, D)),
            full2((1, D)),
            full2((2 * D, D)),
            full2((1, D)),
        ],
        out_specs=pl.BlockSpec((1, 1, D), lambda b: (b, 0, 0)),
        out_shape=jax.ShapeDtypeStruct((B, 1, D), jnp.float32),
        compiler_params=pltpu.CompilerParams(
            dimension_semantics=("arbitrary",)),
    )(S0, oe, ie, Mn, Meo, Mei, Wu, bu2, Wr, br2, Wh, bh2).reshape(B, D)


# speedup vs baseline: 1.1411x; 1.0132x over previous
import functools

import jax
import jax.numpy as jnp
from jax import lax
from jax.experimental import pallas as pl
from jax.experimental.pallas import tpu as pltpu
from jax.experimental.pallas import tpu_sc as plsc

B, N, EPN, D = 8, 512, 32, 256
STEPS = 5

_NC, _NS = 2, 16
_NW = _NC * _NS
_RC = (B * N) // _NW
_L = 16


def _sc_build_body(anT_hbm, aeT_hbm, mn_hbm, meo_hbm, mei_hbm,
                   idxn_v, idxe_v, buf_v):
    wid = lax.axis_index("s") * _NC + lax.axis_index("c")
    base = wid * _RC
    pltpu.sync_copy(anT_hbm.at[:, pl.ds(base, _RC)], idxn_v)
    pltpu.sync_copy(aeT_hbm.at[:, pl.ds(base, _RC)], idxe_v)

    zero16 = jnp.zeros((_L,), jnp.float32)

    def zrow(i, carry):
        for j in range(N // _L):
            buf_v[i, pl.ds(j * _L, _L)] = zero16
        return carry

    lax.fori_loop(0, _RC, zrow, 0)

    lane = lax.iota(jnp.int32, _L)

    def scatter(idx_v, val, kind):
        def body(g, carry):
            row = lane + g * _L
            for k in range(EPN):
                idx = idx_v[k, pl.ds(g * _L, _L)]
                if kind == 0:
                    mask = (idx != 0) & (idx < N)
                    col = idx
                else:
                    mask = idx >= N
                    col = idx - N
                plsc.addupdate_scatter(buf_v, [row, col], val, mask=mask)
            return carry
        lax.fori_loop(0, _RC // _L, body, 0)

    ones = jnp.full((_L,), 1.0, jnp.float32)
    negs = jnp.full((_L,), -1.0, jnp.float32)

    scatter(idxn_v, ones, 0)
    pltpu.sync_copy(buf_v, mn_hbm.at[pl.ds(base, _RC)])
    scatter(idxn_v, negs, 0)

    scatter(idxe_v, ones, 0)
    pltpu.sync_copy(buf_v, meo_hbm.at[pl.ds(base, _RC)])
    scatter(idxe_v, negs, 0)

    scatter(idxe_v, ones, 1)
    pltpu.sync_copy(buf_v, mei_hbm.at[pl.ds(base, _RC)])


_sc_build = functools.partial(
    pl.kernel,
    out_type=(jax.ShapeDtypeStruct((B * N, N), jnp.float32),) * 3,
    mesh=plsc.VectorSubcoreMesh(core_axis_name="c", subcore_axis_name="s"),
    compiler_params=pltpu.CompilerParams(needs_layout_passes=False),
    scratch_types=[
        pltpu.VMEM((EPN, _RC), jnp.int32),
        pltpu.VMEM((EPN, _RC), jnp.int32),
        pltpu.VMEM((_RC, N), jnp.float32),
    ],
)(_sc_build_body)


def _proj_body(nm_ref, em_ref, an_ref, ae_ref, Wn_ref, bn_ref, We_ref,
               be_ref, Wout_ref, Win_ref, s0_ref, oe_ref, ie_ref):
    f32 = jnp.float32
    bf16 = jnp.bfloat16
    An = an_ref[0]
    Ae = ae_ref[0]
    nodes_mask = (jnp.sum(An, axis=1) != 0).astype(f32)[:, None]
    edges_mask = (jnp.sum(Ae, axis=1) != 0).astype(f32)[:, None]

    def mm(a, b):
        return jnp.dot(a.astype(bf16), b.astype(bf16),
                       preferred_element_type=f32)

    S = jnp.tanh(mm(nm_ref[0], Wn_ref[...]) + bn_ref[...]) * nodes_mask
    row_iota = jax.lax.broadcasted_iota(jnp.int32, (N, 1), 0)
    S = jnp.where(row_iota == 1, 1.0, S)
    e = jnp.tanh(mm(em_ref[0], We_ref[...]) + be_ref[...]) * edges_mask
    s0_ref[...] = S.astype(bf16)[None]
    oe_ref[...] = mm(e, Wout_ref[...]).astype(bf16)[None]
    ie_ref[...] = mm(e, Win_ref[...]).astype(bf16)[None]


def _gru_body(s0_ref, oe_ref, ie_ref, mn_ref, meo_ref, mei_ref,
              Wu_ref, bu_ref, Wr_ref, br_ref, Wh_ref, bh_ref, out_ref):
    f32 = jnp.float32
    bf16 = jnp.bfloat16
    Mn = mn_ref[...].astype(bf16)
    Meo = meo_ref[...].astype(bf16)
    Mei = mei_ref[...].astype(bf16)

    def mm(a, b):
        return jnp.dot(a.astype(bf16), b.astype(bf16),
                       preferred_element_type=f32)

    S = s0_ref[0].astype(f32)
    act_e = mm(Meo, oe_ref[0]) + mm(Mei, ie_ref[0])

    Wu_a, Wu_s = Wu_ref[:D, :], Wu_ref[D:, :]
    Wr_a, Wr_s = Wr_ref[:D, :], Wr_ref[D:, :]
    Wh_a, Wh_s = Wh_ref[:D, :], Wh_ref[D:, :]
    bu = bu_ref[...]
    br = br_ref[...]
    bh = bh_ref[...]

    def sigmoid(x):
        return 0.5 * jnp.tanh(0.5 * x) + 0.5

    for _ in range(STEPS):
        act = mm(Mn, S) + act_e
        u = sigmoid(mm(act, Wu_a) + mm(S, Wu_s) + bu)
        r = sigmoid(mm(act, Wr_a) + mm(S, Wr_s) + br)
        h = jnp.tanh(mm(act, Wh_a) + mm(r * S, Wh_s) + bh)
        S = S + u * (h - S)

    out_ref[...] = S[1, :][None, None, :]


def kernel(nodes_m, edges_m, A_nodes, A_edges, Wn, bn, We, be, Wout, Win,
           Wu, bu, Wr, br, Wh, bh):
    anT = A_nodes.reshape(B * N, EPN).T
    aeT = A_edges.reshape(B * N, EPN).T
    Mn, Meo, Mei = _sc_build(anT, aeT)

    bn2, be2, bu2, br2, bh2 = (x.reshape(1, D) for x in (bn, be, bu, br, bh))
    full2 = lambda shape: pl.BlockSpec(shape, lambda b: (0,) * len(shape))
    per_b3 = lambda d1, d2: pl.BlockSpec((1, d1, d2), lambda b: (b, 0, 0))
    m_spec = pl.BlockSpec((N, N), lambda b: (b, 0))

    S0, oe, ie = pl.pallas_call(
        _proj_body,
        grid=(B,),
        in_specs=[
            per_b3(N, D),
            per_b3(N, D),
            per_b3(N, EPN),
            per_b3(N, EPN),
            full2((D, D)),
            full2((1, D)),
            full2((D, D)),
            full2((1, D)),
            full2((D, D)),
            full2((D, D)),
        ],
        out_specs=[per_b3(N, D)] * 3,
        out_shape=[jax.ShapeDtypeStruct((B, N, D), jnp.bfloat16)] * 3,
        compiler_params=pltpu.CompilerParams(
            dimension_semantics=("arbitrary",)),
    )(nodes_m, edges_m, A_nodes, A_edges, Wn, bn2, We, be2, Wout, Win)

    return pl.pallas_call(
        _gru_body,
        grid=(B,),
        in_specs=[
            per_b3(N, D),
            per_b3(N, D),
            per_b3(N, D),
            m_spec,
            m_spec,
            m_spec,
            full2((2 * D, D)),
            full2((1, D)),
            full2((2 * D, D)),
            full2((1, D)),
            full2((2 * D, D)),
            full2((1, D)),
        ],
        out_specs=pl.BlockSpec((1, 1, D), lambda b: (b, 0, 0)),
        out_shape=jax.ShapeDtypeStruct((B, 1, D), jnp.float32),
        compiler_params=pltpu.CompilerParams(
            dimension_semantics=("arbitrary",)),
    )(S0, oe, ie, Mn, Meo, Mei, Wu, bu2, Wr, br2, Wh, bh2).reshape(B, D)


# speedup vs baseline: 1.1929x; 1.0454x over previous
import functools

import jax
import jax.numpy as jnp
from jax import lax
from jax.experimental import pallas as pl
from jax.experimental.pallas import tpu as pltpu
from jax.experimental.pallas import tpu_sc as plsc

B, N, EPN, D = 8, 512, 32, 256
STEPS = 5

_NC, _NS = 2, 16
_NW = _NC * _NS
_RC = (B * N) // _NW
_L = 16


def _sc_build_body(anT_hbm, aeT_hbm, mn_hbm, meo_hbm, mei_hbm,
                   idxn_v, idxe_v, buf_v):
    wid = lax.axis_index("s") * _NC + lax.axis_index("c")
    base = wid * _RC
    pltpu.sync_copy(anT_hbm.at[:, pl.ds(base, _RC)], idxn_v)
    pltpu.sync_copy(aeT_hbm.at[:, pl.ds(base, _RC)], idxe_v)

    zero16 = jnp.zeros((_L,), jnp.float32)

    def zrow(i, carry):
        for j in range(N // _L):
            buf_v[i, pl.ds(j * _L, _L)] = zero16
        return carry

    lax.fori_loop(0, _RC, zrow, 0)

    lane = lax.iota(jnp.int32, _L)

    def scatter(idx_v, val, kind):
        def body(g, carry):
            row = lane + g * _L
            for k in range(EPN):
                idx = idx_v[k, pl.ds(g * _L, _L)]
                if kind == 0:
                    mask = (idx != 0) & (idx < N)
                    col = idx
                else:
                    mask = idx >= N
                    col = idx - N
                plsc.addupdate_scatter(buf_v, [row, col], val, mask=mask)
            return carry
        lax.fori_loop(0, _RC // _L, body, 0)

    ones = jnp.full((_L,), 1.0, jnp.float32)
    negs = jnp.full((_L,), -1.0, jnp.float32)

    scatter(idxn_v, ones, 0)
    pltpu.sync_copy(buf_v, mn_hbm.at[pl.ds(base, _RC)])
    scatter(idxn_v, negs, 0)

    scatter(idxe_v, ones, 0)
    pltpu.sync_copy(buf_v, meo_hbm.at[pl.ds(base, _RC)])
    scatter(idxe_v, negs, 0)

    scatter(idxe_v, ones, 1)
    pltpu.sync_copy(buf_v, mei_hbm.at[pl.ds(base, _RC)])


_sc_build = functools.partial(
    pl.kernel,
    out_type=(jax.ShapeDtypeStruct((B * N, N), jnp.float32),) * 3,
    mesh=plsc.VectorSubcoreMesh(core_axis_name="c", subcore_axis_name="s"),
    compiler_params=pltpu.CompilerParams(needs_layout_passes=False),
    scratch_types=[
        pltpu.VMEM((EPN, _RC), jnp.int32),
        pltpu.VMEM((EPN, _RC), jnp.int32),
        pltpu.VMEM((_RC, N), jnp.float32),
    ],
)(_sc_build_body)


def _proj_body(nm_ref, em_ref, an_ref, ae_ref, Wn_ref, bn_ref, We_ref,
               be_ref, Wout_ref, Win_ref, s0_ref, oe_ref, ie_ref):
    f32 = jnp.float32
    bf16 = jnp.bfloat16
    An = an_ref[0]
    Ae = ae_ref[0]
    nodes_mask = (jnp.sum(An, axis=1) != 0).astype(f32)[:, None]
    edges_mask = (jnp.sum(Ae, axis=1) != 0).astype(f32)[:, None]

    def mm(a, b):
        return jnp.dot(a.astype(bf16), b.astype(bf16),
                       preferred_element_type=f32)

    S = jnp.tanh(mm(nm_ref[0], Wn_ref[...]) + bn_ref[...]) * nodes_mask
    row_iota = jax.lax.broadcasted_iota(jnp.int32, (N, 1), 0)
    S = jnp.where(row_iota == 1, 1.0, S)
    e = jnp.tanh(mm(em_ref[0], We_ref[...]) + be_ref[...]) * edges_mask
    s0_ref[...] = S.astype(bf16)[None]
    oe_ref[...] = mm(e, Wout_ref[...]).astype(bf16)[None]
    ie_ref[...] = mm(e, Win_ref[...]).astype(bf16)[None]


_GB = 2


def _gru_body(s0_ref, oe_ref, ie_ref, mn_ref, meo_ref, mei_ref,
              Wu_ref, bu_ref, Wr_ref, br_ref, Wh_ref, bh_ref, out_ref):
    f32 = jnp.float32
    bf16 = jnp.bfloat16

    def mm(a, b):
        return jnp.dot(a.astype(bf16), b.astype(bf16),
                       preferred_element_type=f32)

    Mn = [mn_ref[pl.ds(g * N, N), :].astype(bf16) for g in range(_GB)]
    act_e = jnp.concatenate(
        [mm(meo_ref[pl.ds(g * N, N), :], oe_ref[g]) +
         mm(mei_ref[pl.ds(g * N, N), :], ie_ref[g]) for g in range(_GB)],
        axis=0)
    S = jnp.concatenate([s0_ref[g].astype(f32) for g in range(_GB)], axis=0)

    Wu_a, Wu_s = Wu_ref[:D, :], Wu_ref[D:, :]
    Wr_a, Wr_s = Wr_ref[:D, :], Wr_ref[D:, :]
    Wh_a, Wh_s = Wh_ref[:D, :], Wh_ref[D:, :]
    bu = bu_ref[...]
    br = br_ref[...]
    bh = bh_ref[...]

    def sigmoid(x):
        return 0.5 * jnp.tanh(0.5 * x) + 0.5

    for _ in range(STEPS):
        act = jnp.concatenate(
            [mm(Mn[g], S[g * N:(g + 1) * N, :]) for g in range(_GB)],
            axis=0) + act_e
        u = sigmoid(mm(act, Wu_a) + mm(S, Wu_s) + bu)
        r = sigmoid(mm(act, Wr_a) + mm(S, Wr_s) + br)
        h = jnp.tanh(mm(act, Wh_a) + mm(r * S, Wh_s) + bh)
        S = S + u * (h - S)

    out_ref[...] = jnp.concatenate(
        [S[g * N + 1, :][None, None, :] for g in range(_GB)], axis=0)


def kernel(nodes_m, edges_m, A_nodes, A_edges, Wn, bn, We, be, Wout, Win,
           Wu, bu, Wr, br, Wh, bh):
    anT = A_nodes.reshape(B * N, EPN).T
    aeT = A_edges.reshape(B * N, EPN).T
    Mn, Meo, Mei = _sc_build(anT, aeT)

    bn2, be2, bu2, br2, bh2 = (x.reshape(1, D) for x in (bn, be, bu, br, bh))
    full2 = lambda shape: pl.BlockSpec(shape, lambda b: (0,) * len(shape))
    per_b3 = lambda d1, d2: pl.BlockSpec((1, d1, d2), lambda b: (b, 0, 0))
    m_spec = pl.BlockSpec((N, N), lambda b: (b, 0))

    S0, oe, ie = pl.pallas_call(
        _proj_body,
        grid=(B,),
        in_specs=[
            per_b3(N, D),
            per_b3(N, D),
            per_b3(N, EPN),
            per_b3(N, EPN),
            full2((D, D)),
            full2((1, D)),
            full2((D, D)),
            full2((1, D)),
            full2((D, D)),
            full2((D, D)),
        ],
        out_specs=[per_b3(N, D)] * 3,
        out_shape=[jax.ShapeDtypeStruct((B, N, D), jnp.bfloat16)] * 3,
        compiler_params=pltpu.CompilerParams(
            dimension_semantics=("arbitrary",)),
    )(nodes_m, edges_m, A_nodes, A_edges, Wn, bn2, We, be2, Wout, Win)

    gb3 = lambda d1, d2: pl.BlockSpec((_GB, d1, d2), lambda b: (b, 0, 0))
    mg_spec = pl.BlockSpec((_GB * N, N), lambda b: (b, 0))
    return pl.pallas_call(
        _gru_body,
        grid=(B // _GB,),
        in_specs=[
            gb3(N, D),
            gb3(N, D),
            gb3(N, D),
            mg_spec,
            mg_spec,
            mg_spec,
            full2((2 * D, D)),
            full2((1, D)),
            full2((2 * D, D)),
            full2((1, D)),
            full2((2 * D, D)),
            full2((1, D)),
        ],
        out_specs=pl.BlockSpec((_GB, 1, D), lambda b: (b, 0, 0)),
        out_shape=jax.ShapeDtypeStruct((B, 1, D), jnp.float32),
        compiler_params=pltpu.CompilerParams(
            dimension_semantics=("arbitrary",)),
    )(S0, oe, ie, Mn, Meo, Mei, Wu, bu2, Wr, br2, Wh, bh2).reshape(B, D)
